# Initial kernel scaffold; baseline (speedup 1.0000x reference)
import jax
import jax.numpy as jnp
from jax.experimental import pallas as pl


def kernel(y, yh, mask):
    raise NotImplementedError("write your pallas kernel here")


# speedup vs baseline: 27.5131x; 27.5131x over previous
import functools

import jax
import jax.numpy as jnp
from jax.experimental import pallas as pl

_L2 = 1.0
_LINF = 0.02
_K = 2048


def _body(y_ref, yh_ref, mask_ref, out_ref):
    B, N = y_ref.shape
    m = mask_ref[...]
    d = y_ref[...] * m - yh_ref[...] * m
    sq = d * d
    total = jnp.sum(sq)

    bits = jax.lax.bitcast_convert_type(sq, jnp.int32)

    lo = jnp.zeros((B, 1), jnp.int32)
    hi = jnp.full((B, 1), jnp.int32(0x7F800000))

    def step(_, carry):
        lo, hi = carry
        mid = lo + ((hi - lo + 1) >> 1)
        cnt = jnp.sum((bits >= mid).astype(jnp.int32), axis=1, keepdims=True)
        ge = cnt >= _K
        lo = jnp.where(ge, mid, lo)
        hi = jnp.where(ge, hi, mid - 1)
        return lo, hi

    lo, hi = jax.lax.fori_loop(0, 31, step, (lo, hi))
    t_bits = lo
    t = jax.lax.bitcast_convert_type(t_bits, jnp.float32)

    gt = bits > t_bits
    s_gt = jnp.sum(jnp.where(gt, sq, 0.0), axis=1, keepdims=True)
    c_gt = jnp.sum(gt.astype(jnp.int32), axis=1, keepdims=True)
    topk_sum = s_gt + (_K - c_gt).astype(jnp.float32) * t

    linf = jnp.sum(topk_sum) / B
    l2 = total / (B * N)
    out_ref[...] = jnp.reshape(_L2 * l2 + _LINF * linf, (1, 1))


@jax.jit
def kernel(y, yh, mask):
    res = pl.pallas_call(
        _body,
        out_shape=jax.ShapeDtypeStruct((1, 1), jnp.float32),
    )(y, yh, mask)
    return res[0, 0]
